# baseline (speedup 1.0000x reference)
import jax
import jax.numpy as jnp
from jax import lax
from jax.experimental import pallas as pl
from jax.experimental.pallas import tpu as pltpu
from jax.experimental.pallas import tpu_sc as plsc

NCHUNK = 4
TOTB = 16384
B = TOTB // NCHUNK
S = 200
V = 1000
D = 128
VP = 1024
NK = VP // 128
NC = 2
NS = 16
L = 16
NW = NC * NS
RPW = B // NW
NG = RPW // L
SEQ_UNROLL = 8


SP = 201
AP = 1041


def _hist_body(tok_hbm, counts_hbm, tok_v, acc_v, si0, si1, so0, so1):
    c = lax.axis_index("c")
    s = lax.axis_index("s")
    wid = s * NC + c
    base = wid * RPW
    lanes = lax.iota(jnp.int32, L)
    zeros16 = jnp.zeros((L,), jnp.float32)
    ones16 = jnp.ones((L,), jnp.float32)
    sem_in = (si0, si1)
    sem_out = (so0, so1)

    def start_in(g):
        row0 = base + g * L
        return pltpu.async_copy(
            tok_hbm.at[pl.ds(row0, L)],
            tok_v.at[g % 2, :, pl.ds(0, S)],
            sem_in[g % 2],
        )

    def start_outs(g):
        row0 = base + g * L
        return [
            pltpu.async_copy(
                acc_v.at[g % 2, :, pl.ds(k * 128, 128)],
                counts_hbm.at[pl.ds(k * B + row0, L)],
                sem_out[g % 2],
            )
            for k in range(NK)
        ]

    in_h = {0: start_in(0)}
    out_h = {}
    for g in range(NG):
        p = g % 2
        in_h[g].wait()
        if g + 1 < NG:
            in_h[g + 1] = start_in(g + 1)
        if g >= 2:
            for h in out_h[g - 2]:
                h.wait()
        tv = tok_v.at[p]
        av = acc_v.at[p]

        def zero_chunk(j, c2, av=av):
            for r in range(L):
                av[r, pl.ds(j * L, L)] = zeros16
            return c2

        lax.fori_loop(0, VP // L, zero_chunk, 0)

        def seq_step(t, c2, tv=tv, av=av):
            for u in range(SEQ_UNROLL):
                pos = jnp.full((L,), t * SEQ_UNROLL + u, jnp.int32)
                tok = plsc.load_gather(tv, [lanes, pos])
                plsc.addupdate_scatter(av, [lanes, tok], ones16)
            return c2

        lax.fori_loop(0, S // SEQ_UNROLL, seq_step, 0)
        out_h[g] = start_outs(g)
    for g in (NG - 2, NG - 1):
        for h in out_h[g]:
            h.wait()


_hist = pl.kernel(
    _hist_body,
    out_type=jax.ShapeDtypeStruct((NK * B, 128), jnp.float32),
    mesh=plsc.VectorSubcoreMesh(
        core_axis_name="c", subcore_axis_name="s", num_cores=NC, num_subcores=NS
    ),
    scratch_types=[
        pltpu.VMEM((2, L, SP), jnp.int32),
        pltpu.VMEM((2, L, AP), jnp.float32),
        pltpu.SemaphoreType.DMA,
        pltpu.SemaphoreType.DMA,
        pltpu.SemaphoreType.DMA,
        pltpu.SemaphoreType.DMA,
    ],
    compiler_params=pltpu.CompilerParams(
        use_tc_tiling_on_sc=False, needs_layout_passes=False
    ),
)


def _matmul_body(counts_ref, table_ref, out_prev_ref, out_ref):
    k = pl.program_id(1)

    @pl.when(k == 0)
    def _init():
        out_ref[...] = jnp.zeros_like(out_ref)

    out_ref[...] += jnp.dot(
        counts_ref[...].astype(jnp.bfloat16),
        table_ref[...],
        preferred_element_type=jnp.float32,
    )

    @pl.when(k == NK - 1)
    def _scale():
        out_ref[...] *= 1.0 / S


BM = 2048


def _pooled_matmul(counts, table_bf16, out_prev, ci):
    return pl.pallas_call(
        _matmul_body,
        grid=(B // BM, NK),
        in_specs=[
            pl.BlockSpec((BM, 128), lambda i, k: (k * (B // BM) + i, 0)),
            pl.BlockSpec((128, D), lambda i, k: (k, 0)),
            pl.BlockSpec(memory_space=pl.ANY),
        ],
        out_specs=pl.BlockSpec(
            (BM, D), lambda i, k, ci=ci: (ci * (B // BM) + i, 0)
        ),
        out_shape=jax.ShapeDtypeStruct((TOTB, D), jnp.float32),
        input_output_aliases={2: 0},
    )(counts, table_bf16, out_prev)


def kernel(tokens, table):
    tokens = tokens.astype(jnp.int32)
    table_bf16 = jnp.concatenate(
        [table, jnp.zeros((VP - V, D), table.dtype)], axis=0
    ).astype(jnp.bfloat16)
    out = jnp.zeros((TOTB, D), jnp.float32)
    for ci in range(NCHUNK):
        tok_c = tokens[ci * B:(ci + 1) * B]
        counts = _hist(tok_c)
        out = _pooled_matmul(counts, table_bf16, out, ci)
    return out


# speedup vs baseline: 1.4014x; 1.4014x over previous
import jax
import jax.numpy as jnp
from jax import lax
from jax.experimental import pallas as pl
from jax.experimental.pallas import tpu as pltpu
from jax.experimental.pallas import tpu_sc as plsc

NCHUNK = 4
TOTB = 16384
B = TOTB // NCHUNK
S = 200
V = 1000
D = 128
VP = 1024
NK = VP // 128
NC = 2
NS = 16
L = 16
NW = NC * NS
RPW = B // NW
NG = RPW // L
SEQ_UNROLL = 8


def _hist_body(tok_hbm, counts_hbm, tok_v, acc_v, si0, si1, so0, so1):
    c = lax.axis_index("c")
    s = lax.axis_index("s")
    wid = s * NC + c
    base = wid * RPW
    lanes = lax.iota(jnp.int32, L)
    lanes_s = lanes * S
    zeros16 = jnp.zeros((L,), jnp.float32)
    ones16 = jnp.ones((L,), jnp.float32)
    sem_in = (si0, si1)
    sem_out = (so0, so1)

    def start_in(g):
        row0 = base + g * L
        return pltpu.async_copy(
            tok_hbm.at[pl.ds(row0 * S, L * S)], tok_v.at[g % 2], sem_in[g % 2]
        )

    def start_outs(g):
        row0 = base + g * L
        return [
            pltpu.async_copy(
                acc_v.at[g % 2, k],
                counts_hbm.at[pl.ds(k * B + row0, L)],
                sem_out[g % 2],
            )
            for k in range(NK)
        ]

    in_h = {0: start_in(0)}
    out_h = {}
    for g in range(NG):
        p = g % 2
        in_h[g].wait()
        if g + 1 < NG:
            in_h[g + 1] = start_in(g + 1)
        if g >= 2:
            for h in out_h[g - 2]:
                h.wait()
        tv = tok_v.at[p]
        av = acc_v.at[p]

        def zero_chunk(j, c2, av=av):
            for k in range(NK):
                for r in range(L):
                    av[k, r, pl.ds(j * L, L)] = zeros16
            return c2

        lax.fori_loop(0, 128 // L, zero_chunk, 0)

        @plsc.parallel_loop(0, S, step=1, unroll=SEQ_UNROLL)
        def seq_step(t, tv=tv, av=av):
            tok = plsc.load_gather(tv, [lanes_s + t])
            plsc.addupdate_scatter(
                av,
                [lax.shift_right_logical(tok, 7), lanes,
                 lax.bitwise_and(tok, 127)],
                ones16,
            )

        out_h[g] = start_outs(g)
    for g in (NG - 2, NG - 1):
        for h in out_h[g]:
            h.wait()


_hist = pl.kernel(
    _hist_body,
    out_type=jax.ShapeDtypeStruct((NK * B, 128), jnp.float32),
    mesh=plsc.VectorSubcoreMesh(
        core_axis_name="c", subcore_axis_name="s", num_cores=NC, num_subcores=NS
    ),
    scratch_types=[
        pltpu.VMEM((2, L * S), jnp.int32),
        pltpu.VMEM((2, NK, L, 128), jnp.float32),
        pltpu.SemaphoreType.DMA,
        pltpu.SemaphoreType.DMA,
        pltpu.SemaphoreType.DMA,
        pltpu.SemaphoreType.DMA,
    ],
    compiler_params=pltpu.CompilerParams(
        use_tc_tiling_on_sc=False, needs_layout_passes=False
    ),
)


def _matmul_body(counts_ref, table_ref, out_prev_ref, out_ref):
    k = pl.program_id(1)

    @pl.when(k == 0)
    def _init():
        out_ref[...] = jnp.zeros_like(out_ref)

    out_ref[...] += jnp.dot(
        counts_ref[...].astype(jnp.bfloat16),
        table_ref[...],
        preferred_element_type=jnp.float32,
    )

    @pl.when(k == NK - 1)
    def _scale():
        out_ref[...] *= 1.0 / S


BM = 2048


def _pooled_matmul(counts, table_bf16, out_prev, ci):
    return pl.pallas_call(
        _matmul_body,
        grid=(B // BM, NK),
        in_specs=[
            pl.BlockSpec((BM, 128), lambda i, k: (k * (B // BM) + i, 0)),
            pl.BlockSpec((128, D), lambda i, k: (k, 0)),
            pl.BlockSpec(memory_space=pl.ANY),
        ],
        out_specs=pl.BlockSpec(
            (BM, D), lambda i, k, ci=ci: (ci * (B // BM) + i, 0)
        ),
        out_shape=jax.ShapeDtypeStruct((TOTB, D), jnp.float32),
        input_output_aliases={2: 0},
    )(counts, table_bf16, out_prev)


def kernel(tokens, table):
    tokens = tokens.astype(jnp.int32)
    table_bf16 = jnp.concatenate(
        [table, jnp.zeros((VP - V, D), table.dtype)], axis=0
    ).astype(jnp.bfloat16)
    out = jnp.zeros((TOTB, D), jnp.float32)
    for ci in range(NCHUNK):
        tok_c = tokens[ci * B:(ci + 1) * B].reshape(-1)
        counts = _hist(tok_c)
        out = _pooled_matmul(counts, table_bf16, out, ci)
    return out
